# baseline (speedup 1.0000x reference)
import functools

import jax
import jax.numpy as jnp
from jax import lax
from jax.experimental import pallas as pl
from jax.experimental.pallas import tpu as pltpu
from jax.experimental.pallas import tpu_sc as plsc

D_MODEL = 1024
SCALE = 32.0
NC, NS, L = 2, 16, 16
NW = NC * NS
B = 4 * 8192
BPW = B // NW
CH = 64
NCHUNK = BPW // CH
VPR = D_MODEL // L

_mesh = plsc.VectorSubcoreMesh(core_axis_name="c", subcore_axis_name="s")


@functools.partial(
    pl.kernel,
    out_type=jax.ShapeDtypeStruct((B, D_MODEL), jnp.float32),
    mesh=_mesh,
    scratch_types=[
        pltpu.VMEM((BPW,), jnp.int32),
        pltpu.VMEM((CH, D_MODEL), jnp.float32),
        pltpu.SemaphoreType.DMA,
    ],
)
def _embed_sc(x_hbm, table_hbm, out_hbm, idx_v, rows_v, sem):
    wid = lax.axis_index("s") * NC + lax.axis_index("c")
    base = wid * BPW
    pltpu.sync_copy(x_hbm.at[pl.ds(base, BPW)], idx_v)

    def chunk_body(c, carry):
        off = pl.multiple_of(c * CH, 8)
        pltpu.async_copy(
            table_hbm.at[idx_v.at[pl.ds(off, CH)]], rows_v, sem
        ).wait()

        def scale_body(i, carry2):
            r = i // VPR
            col = (i % VPR) * L
            rows_v[r, pl.ds(col, L)] = rows_v[r, pl.ds(col, L)] * SCALE
            return carry2

        lax.fori_loop(0, CH * VPR, scale_body, 0)
        pltpu.sync_copy(rows_v, out_hbm.at[pl.ds(base + off, CH)])
        return carry

    lax.fori_loop(0, NCHUNK, chunk_body, 0)


def kernel(x, embedding):
    xf = x.reshape(-1).astype(jnp.int32)
    out = _embed_sc(xf, embedding)
    return out.reshape(x.shape[0], x.shape[1], D_MODEL)


# speedup vs baseline: 3.9278x; 3.9278x over previous
import functools

import jax
import jax.numpy as jnp
from jax import lax
from jax.experimental import pallas as pl
from jax.experimental.pallas import tpu as pltpu
from jax.experimental.pallas import tpu_sc as plsc

D_MODEL = 1024
SCALE = 32.0
NC, NS, L = 2, 16, 16
NW = NC * NS
B = 4 * 8192
BPW = B // NW
CH = 32
NCHUNK = BPW // CH
NBUF = 2
VPR = D_MODEL // L

_mesh = plsc.VectorSubcoreMesh(core_axis_name="c", subcore_axis_name="s")


@functools.partial(
    pl.kernel,
    out_type=jax.ShapeDtypeStruct((B, D_MODEL), jnp.float32),
    mesh=_mesh,
    scratch_types=[
        pltpu.VMEM((BPW,), jnp.int32),
        pltpu.VMEM((CH, D_MODEL), jnp.float32),
        pltpu.VMEM((CH, D_MODEL), jnp.float32),
        pltpu.SemaphoreType.DMA,
        pltpu.SemaphoreType.DMA,
        pltpu.SemaphoreType.DMA,
        pltpu.SemaphoreType.DMA,
    ],
)
def _embed_sc(x_hbm, table_hbm, out_hbm, idx_v, buf0, buf1, g0, g1, s0, s1):
    wid = lax.axis_index("s") * NC + lax.axis_index("c")
    base = wid * BPW
    pltpu.sync_copy(x_hbm.at[pl.ds(base, BPW)], idx_v)

    bufs = (buf0, buf1)
    gsems = (g0, g1)
    ssems = (s0, s1)

    def issue_gather(c, b):
        off = pl.multiple_of(c * CH, 8)
        pltpu.async_copy(table_hbm.at[idx_v.at[pl.ds(off, CH)]], bufs[b], gsems[b])

    def wait_gather(b):
        pltpu.make_async_copy(
            table_hbm.at[pl.ds(0, CH)], bufs[b], gsems[b]
        ).wait()

    def scale_buf(b):
        buf = bufs[b]

        @plsc.parallel_loop(0, CH)
        def _(r):
            for j in range(VPR):
                buf[r, pl.ds(j * L, L)] = buf[r, pl.ds(j * L, L)] * SCALE

    def issue_scatter(c, b):
        off = pl.multiple_of(c * CH, 8)
        pltpu.async_copy(bufs[b], out_hbm.at[pl.ds(base + off, CH)], ssems[b])

    def wait_scatter(b):
        pltpu.make_async_copy(
            bufs[b], out_hbm.at[pl.ds(0, CH)], ssems[b]
        ).wait()

    def visit(c, b, prefetch):
        wait_gather(b)
        scale_buf(b)
        issue_scatter(c, b)
        wait_scatter(b)
        if prefetch:
            issue_gather(c + NBUF, b)

    for b in range(NBUF):
        issue_gather(b, b)

    def outer(t, carry):
        for b in range(NBUF):
            visit(t * NBUF + b, b, prefetch=True)
        return carry

    lax.fori_loop(0, NCHUNK // NBUF - 1, outer, 0)
    for b in range(NBUF):
        visit(NCHUNK - NBUF + b, b, prefetch=False)


def kernel(x, embedding):
    xf = x.reshape(-1).astype(jnp.int32)
    out = _embed_sc(xf, embedding)
    return out.reshape(x.shape[0], x.shape[1], D_MODEL)
